# Initial kernel scaffold; baseline (speedup 1.0000x reference)
import jax
import jax.numpy as jnp
from jax.experimental import pallas as pl


def kernel(idx, W0, W1):
    raise NotImplementedError("write your pallas kernel here")


# speedup vs baseline: 5.4824x; 5.4824x over previous
import functools

import jax
import jax.numpy as jnp
from jax import lax
from jax.experimental import pallas as pl
from jax.experimental.pallas import tpu as pltpu
from jax.experimental.pallas import tpu_sc as plsc

NC, NS, LANES = 2, 16, 16
NW = NC * NS


def _make_sc_lookup(R, V, E, CH):
    assert R % (NW * CH) == 0
    RPW = R // NW
    NCHUNK = RPW // CH
    assert CH % (2 * LANES) == 0
    NSTAGE = 5
    assert V % NSTAGE == 0
    VS = V // NSTAGE
    assert VS % 8 == 0 and V % 8 == 0

    mesh = plsc.VectorSubcoreMesh(core_axis_name="c", subcore_axis_name="s")

    @functools.partial(
        pl.kernel,
        out_type=jax.ShapeDtypeStruct((R, E), jnp.float32),
        mesh=mesh,
        scratch_types=[
            pltpu.VMEM((NCHUNK, CH), jnp.int32),
            pltpu.VMEM((CH, E), jnp.float32),
            pltpu.VMEM((CH, E), jnp.float32),
            pltpu.VMEM_SHARED((2 * V, E), jnp.float32),
            pltpu.SemaphoreType.DMA,
            pltpu.SemaphoreType.DMA,
            pltpu.SemaphoreType.DMA,
            pltpu.SemaphoreType.DMA,
        ],
        compiler_params=pltpu.CompilerParams(use_tc_tiling_on_sc=False),
    )
    def lookup(idx_hbm, w0_hbm, w1_hbm, out_hbm, idx_v, rows0_v, rows1_v,
               tbl_sh, gsem0, gsem1, wsem0, wsem1):
        c = lax.axis_index("c")
        s = lax.axis_index("s")
        wid = s * NC + c

        @pl.when(s < NSTAGE)
        def _():
            pltpu.sync_copy(w0_hbm.at[pl.ds(s * VS, VS)],
                            tbl_sh.at[pl.ds(s * VS, VS)])

        @pl.when(jnp.logical_and(s >= NSTAGE, s < 2 * NSTAGE))
        def _():
            s2 = s - NSTAGE
            pltpu.sync_copy(w1_hbm.at[pl.ds(s2 * VS, VS)],
                            tbl_sh.at[pl.ds(V + s2 * VS, VS)])

        pltpu.sync_copy(idx_hbm.at[wid], idx_v)
        pat = (lax.iota(jnp.int32, LANES) & 1) * V

        def build(ci, _):
            for j in range(CH // LANES):
                sl = pl.ds(j * LANES, LANES)
                idx_v[ci, sl] = idx_v[ci, sl] + pat
            return 0

        lax.fori_loop(0, NCHUNK, build, 0)
        plsc.subcore_barrier()

        base = wid * RPW
        rows = (rows0_v, rows1_v)
        gsems = (gsem0, gsem1)
        wsems = (wsem0, wsem1)

        def gather(ci, buf):
            return pltpu.async_copy(tbl_sh.at[idx_v.at[ci]], rows[buf],
                                    gsems[buf])

        def write(ci, buf):
            return pltpu.async_copy(
                rows[buf], out_hbm.at[pl.ds(base + ci * CH, CH)], wsems[buf])

        def wait_write(buf):
            pltpu.make_async_copy(
                rows[buf], out_hbm.at[pl.ds(base, CH)], wsems[buf]).wait()

        assert NCHUNK % 2 == 0 and NCHUNK >= 4
        gather(0, 0).wait()
        write(0, 0)
        gather(1, 1).wait()
        write(1, 1)

        def step(g, _):
            for b in range(2):
                ci = 2 * g + b
                wait_write(b)
                gather(ci, b).wait()
                write(ci, b)
            return 0

        lax.fori_loop(1, NCHUNK // 2, step, 0)
        wait_write(0)
        wait_write(1)

    return lookup


def kernel(idx, W0, W1):
    B, L, NB = idx.shape
    V, E = W0.shape
    assert NB == 2 and W1.shape == (V, E)
    R = B * L * NB
    CH = 128
    idx3 = idx.astype(jnp.int32).reshape(NW, R // (NW * CH), CH)
    out = _make_sc_lookup(R, V, E, CH)(idx3, W0, W1)
    return out.reshape(B, L, NB * E)


# speedup vs baseline: 12.2184x; 2.2286x over previous
import functools

import jax
import jax.numpy as jnp
from jax import lax
from jax.experimental import pallas as pl
from jax.experimental.pallas import tpu as pltpu
from jax.experimental.pallas import tpu_sc as plsc

NC, NS, LANES = 2, 16, 16
NW = NC * NS
NSLOT = 4


def _make_sc_lookup(B, L, V, D, NB_E):
    assert B % NW == 0
    EPW = B // NW
    assert EPW % NB_E == 0
    NCH = EPW // NB_E
    CH = NB_E * L
    assert CH <= 128
    assert NCH % NSLOT == 0 and NCH // NSLOT >= 2
    NSTAGE = 5
    assert V % NSTAGE == 0 and (V // NSTAGE) % 8 == 0
    VS = V // NSTAGE

    mesh = plsc.VectorSubcoreMesh(core_axis_name="c", subcore_axis_name="s")

    @functools.partial(
        pl.kernel,
        out_type=jax.ShapeDtypeStruct((B, L, D), jnp.float32),
        mesh=mesh,
        scratch_types=[
            pltpu.VMEM((NCH, CH), jnp.int32),
            pltpu.VMEM((NCH, CH), jnp.int32),
            *[pltpu.VMEM((CH, D), jnp.float32) for _ in range(NSLOT)],
            pltpu.VMEM_SHARED((V, D), jnp.float32),
            pltpu.VMEM_SHARED((V, D), jnp.float32),
            *[pltpu.SemaphoreType.DMA for _ in range(2 * NSLOT)],
        ],
    )
    def lookup(i0_hbm, i1_hbm, w0z_hbm, w1z_hbm, out_hbm,
               i0_v, i1_v, *rest):
        bufs = rest[:NSLOT]
        t0_sh, t1_sh = rest[NSLOT], rest[NSLOT + 1]
        gsems = rest[NSLOT + 2:2 * NSLOT + 2]
        wsems = rest[2 * NSLOT + 2:]
        c = lax.axis_index("c")
        s = lax.axis_index("s")
        wid = s * NC + c

        @pl.when(s < NSTAGE)
        def _():
            pltpu.sync_copy(w0z_hbm.at[pl.ds(s * VS, VS)],
                            t0_sh.at[pl.ds(s * VS, VS)])

        @pl.when(jnp.logical_and(s >= NSTAGE, s < 2 * NSTAGE))
        def _():
            s2 = s - NSTAGE
            pltpu.sync_copy(w1z_hbm.at[pl.ds(s2 * VS, VS)],
                            t1_sh.at[pl.ds(s2 * VS, VS)])

        pltpu.sync_copy(i0_hbm.at[wid], i0_v)
        pltpu.sync_copy(i1_hbm.at[wid], i1_v)
        plsc.subcore_barrier()

        ebase = wid * EPW

        def startA(ci, b):
            return pltpu.async_copy(t0_sh.at[i0_v.at[ci]], bufs[b], gsems[b])

        def startB(ci, b):
            return pltpu.async_copy(t1_sh.at[i1_v.at[ci]], bufs[b], gsems[b],
                                    add=True)

        def startW(ci, b):
            return pltpu.async_copy(
                bufs[b].reshape(NB_E, L, D),
                out_hbm.at[pl.ds(ebase + ci * NB_E, NB_E)], wsems[b])

        def waitG(b):
            pltpu.make_async_copy(t0_sh.at[i0_v.at[0]], bufs[b],
                                  gsems[b]).wait()

        def waitW(b):
            pltpu.make_async_copy(
                bufs[b].reshape(NB_E, L, D),
                out_hbm.at[pl.ds(ebase, NB_E)], wsems[b]).wait()

        for b in range(NSLOT):
            startA(b, b)

        def group(g, _):
            for b in range(NSLOT):
                ci = g * NSLOT + b
                waitG(b)
                startB(ci, b)
            for b in range(NSLOT):
                ci = g * NSLOT + b
                waitG(b)
                startW(ci, b)
            for b in range(NSLOT):
                ci2 = (g + 1) * NSLOT + b
                @pl.when(ci2 < NCH)
                def _():
                    waitW(b)
                    startA(ci2, b)
            return 0

        lax.fori_loop(0, NCH // NSLOT, group, 0)
        for b in range(NSLOT):
            waitW(b)

    return lookup


def kernel(idx, W0, W1):
    B, L, NB = idx.shape
    V, E = W0.shape
    assert NB == 2 and W1.shape == (V, E)
    D = NB * E
    NB_E = 2
    idx32 = idx.astype(jnp.int32)
    i0 = idx32[..., 0].reshape(NW, B // NW // NB_E, NB_E * L)
    i1 = idx32[..., 1].reshape(NW, B // NW // NB_E, NB_E * L)
    w0z = jnp.pad(W0, ((0, 0), (0, E)))
    w1z = jnp.pad(W1, ((0, 0), (E, 0)))
    return _make_sc_lookup(B, L, V, D, NB_E)(i0, i1, w0z, w1z)


# speedup vs baseline: 20.3329x; 1.6641x over previous
import functools

import jax
import jax.numpy as jnp
from jax import lax
from jax.experimental import pallas as pl
from jax.experimental.pallas import tpu as pltpu
from jax.experimental.pallas import tpu_sc as plsc

NC, NS, LANES = 2, 16, 16
NW = NC * NS
NSLOT = 5
CH = 128


def _make_sc_lookup(R, V, D):
    assert R % (NW * CH) == 0
    RPW = R // NW
    NCH = RPW // CH
    assert NCH % NSLOT == 0 and NCH // NSLOT >= 2
    NSTAGE = 5
    assert V % NSTAGE == 0 and (V // NSTAGE) % 8 == 0
    VS = V // NSTAGE

    mesh = plsc.VectorSubcoreMesh(core_axis_name="c", subcore_axis_name="s")

    @functools.partial(
        pl.kernel,
        out_type=jax.ShapeDtypeStruct((R, D), jnp.float32),
        mesh=mesh,
        scratch_types=[
            pltpu.VMEM((NCH, CH), jnp.int32),
            pltpu.VMEM((NCH, CH), jnp.int32),
            *[pltpu.VMEM((CH, D), jnp.float32) for _ in range(NSLOT)],
            pltpu.VMEM_SHARED((V, D), jnp.float32),
            pltpu.VMEM_SHARED((V, D), jnp.float32),
            *[pltpu.SemaphoreType.DMA for _ in range(2 * NSLOT)],
        ],
    )
    def lookup(i0_hbm, i1_hbm, w0z_hbm, w1z_hbm, out_hbm,
               i0_v, i1_v, *rest):
        bufs = rest[:NSLOT]
        t0_sh, t1_sh = rest[NSLOT], rest[NSLOT + 1]
        gsems = rest[NSLOT + 2:2 * NSLOT + 2]
        wsems = rest[2 * NSLOT + 2:]
        c = lax.axis_index("c")
        s = lax.axis_index("s")
        wid = s * NC + c

        @pl.when(s < NSTAGE)
        def _():
            pltpu.sync_copy(w0z_hbm.at[pl.ds(s * VS, VS)],
                            t0_sh.at[pl.ds(s * VS, VS)])

        @pl.when(jnp.logical_and(s >= NSTAGE, s < 2 * NSTAGE))
        def _():
            s2 = s - NSTAGE
            pltpu.sync_copy(w1z_hbm.at[pl.ds(s2 * VS, VS)],
                            t1_sh.at[pl.ds(s2 * VS, VS)])

        pltpu.sync_copy(i0_hbm.at[wid], i0_v)
        pltpu.sync_copy(i1_hbm.at[wid], i1_v)
        plsc.subcore_barrier()

        rbase = wid * RPW

        def startA(ci, b):
            pltpu.async_copy(t0_sh.at[i0_v.at[ci]], bufs[b], gsems[b])

        def startB(ci, b):
            pltpu.async_copy(t1_sh.at[i1_v.at[ci]], bufs[b], gsems[b],
                             add=True)

        def startW(ci, b):
            pltpu.async_copy(bufs[b], out_hbm.at[pl.ds(rbase + ci * CH, CH)],
                             wsems[b])

        def waitG(b):
            pltpu.make_async_copy(t0_sh.at[i0_v.at[0]], bufs[b],
                                  gsems[b]).wait()

        def waitW(b):
            pltpu.make_async_copy(bufs[b], out_hbm.at[pl.ds(rbase, CH)],
                                  wsems[b]).wait()

        for b in range(NSLOT):
            startA(b, b)

        def group(g, _):
            for b in range(NSLOT):
                ci = g * NSLOT + b
                waitG(b)
                startB(ci, b)
            for b in range(NSLOT):
                ci = g * NSLOT + b
                waitG(b)
                startW(ci, b)
            for b in range(NSLOT):
                ci2 = (g + 1) * NSLOT + b
                @pl.when(ci2 < NCH)
                def _():
                    waitW(b)
                    startA(ci2, b)
            return 0

        lax.fori_loop(0, NCH // NSLOT, group, 0)
        for b in range(NSLOT):
            waitW(b)

    return lookup


def kernel(idx, W0, W1):
    B, L, NB = idx.shape
    V, E = W0.shape
    assert NB == 2 and W1.shape == (V, E)
    D = NB * E
    R = B * L
    idx32 = idx.astype(jnp.int32)
    i0 = idx32[..., 0].T.reshape(NW, R // (NW * CH), CH)
    i1 = idx32[..., 1].T.reshape(NW, R // (NW * CH), CH)
    w0z = jnp.pad(W0, ((0, 0), (0, E)))
    w1z = jnp.pad(W1, ((0, 0), (E, 0)))
    out = _make_sc_lookup(R, V, D)(i0, i1, w0z, w1z)
    return out.reshape(L, B, D).transpose(1, 0, 2)
